# Initial kernel scaffold; baseline (speedup 1.0000x reference)
import jax
import jax.numpy as jnp
from jax.experimental import pallas as pl


def kernel(hidden_states, gate_proj_w, gate_weights, up_weights, down_weights):
    raise NotImplementedError("write your pallas kernel here")


# speedup vs baseline: 1.6429x; 1.6429x over previous
import jax
import jax.numpy as jnp
from jax.experimental import pallas as pl

T = 2048
H = 1024
I = 1024
E = 8
TILE_T = 512
T_TILES = T // TILE_T


def _moe_dense_kernel(x_ref, gw_ref, wg_ref, wu_ref, wd_ref, out_ref):
    e = pl.program_id(0)
    t = pl.program_id(1)

    x = x_ref[pl.ds(t * TILE_T, TILE_T), :]

    logits = jax.lax.dot_general(
        x, gw_ref[...], (((1,), (1,)), ((), ())),
        preferred_element_type=jnp.float32)
    a1 = jnp.argmax(logits, axis=1)
    l1 = jnp.max(logits, axis=1)
    cols = jax.lax.broadcasted_iota(jnp.int32, (TILE_T, E), 1)
    masked = jnp.where(cols == a1[:, None], -jnp.inf, logits)
    a2 = jnp.argmax(masked, axis=1)
    l2 = jnp.max(masked, axis=1)
    w1 = jax.nn.sigmoid(l1 - l2)
    w2 = 1.0 - w1
    coef = jnp.where(a1 == e, w1, jnp.where(a2 == e, w2, 0.0))

    xb = x.astype(jnp.bfloat16)
    wg = wg_ref[0].astype(jnp.bfloat16)
    wu = wu_ref[0].astype(jnp.bfloat16)
    wd = wd_ref[0].astype(jnp.bfloat16)
    g = jnp.dot(xb, wg, preferred_element_type=jnp.float32)
    u = jnp.dot(xb, wu, preferred_element_type=jnp.float32)
    inter = (g * jax.nn.sigmoid(g) * u).astype(jnp.bfloat16)
    d = jnp.dot(inter, wd, preferred_element_type=jnp.float32)
    contrib = coef[:, None] * d

    @pl.when(e == 0)
    def _init():
        out_ref[pl.ds(t * TILE_T, TILE_T), :] = contrib

    @pl.when(e != 0)
    def _acc():
        out_ref[pl.ds(t * TILE_T, TILE_T), :] += contrib


def kernel(hidden_states, gate_proj_w, gate_weights, up_weights, down_weights):
    return pl.pallas_call(
        _moe_dense_kernel,
        grid=(E, T_TILES),
        in_specs=[
            pl.BlockSpec((T, H), lambda e, t: (0, 0)),
            pl.BlockSpec((E, H), lambda e, t: (0, 0)),
            pl.BlockSpec((1, H, I), lambda e, t: (e, 0, 0)),
            pl.BlockSpec((1, H, I), lambda e, t: (e, 0, 0)),
            pl.BlockSpec((1, I, H), lambda e, t: (e, 0, 0)),
        ],
        out_specs=pl.BlockSpec((T, H), lambda e, t: (0, 0)),
        out_shape=jax.ShapeDtypeStruct((T, H), jnp.float32),
    )(hidden_states, gate_proj_w, gate_weights, up_weights, down_weights)
